# Initial kernel scaffold; baseline (speedup 1.0000x reference)
import jax
import jax.numpy as jnp
from jax.experimental import pallas as pl


def kernel(x, edge_index, batch, root_index, W1, b1, W2, b2, W3, b3, Wc1, bc1, Wc2, bc2):
    raise NotImplementedError("write your pallas kernel here")


# speedup vs baseline: 8.7336x; 8.7336x over previous
import functools

import jax
import jax.numpy as jnp
from jax import lax
from jax.experimental import pallas as pl
from jax.experimental.pallas import tpu as pltpu
from jax.experimental.pallas import tpu_sc as plsc

N = 100000
E = 1600000
G = 128
NC, NS, L = 2, 16, 16
NP = 100352

BK = 100
EROWS = E // BK
RPT = EROWS // NS
NB = 50
NCH = RPT // NB

RPS = NP // NS
ZR = 98

NW = NC * NS
EPT = E // NW
DCH = 2000
PW = 896

NBLK = 512
NGRID = NP // NBLK


def _sc_mesh():
  return plsc.VectorSubcoreMesh(
      core_axis_name="c", subcore_axis_name="s",
      num_cores=NC, num_subcores=NS)


def _deg_body(dst_hbm, out_hbm, part_hbm, hist_v, idx_v, piece_v, res_v):
  cid = lax.axis_index("c")
  sid = lax.axis_index("s")
  wid = sid * NC + cid

  def zero_body(i, carry):
    hist_v[pl.ds(i * L, L)] = jnp.zeros((L,), jnp.float32)
    return carry
  lax.fori_loop(0, NP // L, zero_body, 0)

  ones = jnp.ones((L,), jnp.float32)
  base = wid * EPT

  def chunk_body(j, carry):
    pltpu.sync_copy(dst_hbm.at[pl.ds(base + j * DCH, DCH)], idx_v)

    def scat_body(k, c2):
      idx = idx_v[pl.ds(k * L, L)]
      plsc.addupdate_scatter(hist_v, [idx], ones)
      return c2
    lax.fori_loop(0, DCH // L, scat_body, 0)
    return carry
  lax.fori_loop(0, EPT // DCH, chunk_body, 0)

  pltpu.sync_copy(hist_v, part_hbm.at[wid])
  plsc.subcore_barrier()

  colbase = sid * RPS
  for p in range(7):
    for r in range(NS):
      pltpu.sync_copy(
          part_hbm.at[r * NC + cid, pl.ds(colbase + p * PW, PW)],
          piece_v.at[r])

    def red_body(k, carry):
      sl = pl.ds(k * L, L)
      acc = piece_v[0, sl]
      for r in range(1, NS):
        acc = acc + piece_v[r, sl]
      res_v[sl] = acc
      return carry
    lax.fori_loop(0, PW // L, red_body, 0)
    pltpu.sync_copy(res_v, out_hbm.at[cid, pl.ds(colbase + p * PW, PW)])


_SC_PARAMS = pltpu.CompilerParams(
    use_tc_tiling_on_sc=False, needs_layout_passes=False)

_deg_kernel = functools.partial(
    pl.kernel,
    out_type=[
        jax.ShapeDtypeStruct((NC, NP), jnp.float32),
        jax.ShapeDtypeStruct((NW, NP), jnp.float32),
    ],
    mesh=_sc_mesh(),
    compiler_params=_SC_PARAMS,
    scratch_types=[
        pltpu.VMEM((NP,), jnp.float32),
        pltpu.VMEM((DCH,), jnp.int32),
        pltpu.VMEM((NS, PW), jnp.float32),
        pltpu.VMEM((PW,), jnp.float32),
    ],
)(_deg_body)


def _conv_body(g4_hbm, src4q_hbm, dst_hbm, ridx_hbm, xtab_hbm,
               s_out, rv_out, acc, srcq_v, dst_v, rows_v, zbuf_v,
               ridx_v, rrows_v):
  cid = lax.axis_index("c")
  sid = lax.axis_index("s")

  @pl.when(jnp.logical_and(cid == 0, sid == 0))
  def _root_gather():
    pltpu.sync_copy(ridx_hbm, ridx_v)
    pltpu.sync_copy(xtab_hbm.at[ridx_v], rrows_v)
    pltpu.sync_copy(rrows_v, rv_out)

  def zb_body(i, carry):
    zbuf_v[i, :] = jnp.zeros((L,), jnp.float32)
    return carry
  lax.fori_loop(0, ZR, zb_body, 0)

  for t in range(2):
    q = 2 * cid + t

    def z_body(z, carry):
      pltpu.sync_copy(zbuf_v, acc.at[pl.ds(sid * RPS + z * ZR, ZR)])
      return carry
    lax.fori_loop(0, RPS // ZR, z_body, 0)
    plsc.subcore_barrier()

    def chunk_body(j, carry):
      r0 = sid * RPT + j * NB
      pltpu.sync_copy(src4q_hbm.at[q, pl.ds(r0, NB)], srcq_v)
      pltpu.sync_copy(dst_hbm.at[pl.ds(r0, NB)], dst_v)

      def blk_body(j2, c2):
        pltpu.sync_copy(g4_hbm.at[srcq_v.at[j2]], rows_v)
        pltpu.sync_copy(rows_v, acc.at[dst_v.at[j2]], add=True)
        return c2
      lax.fori_loop(0, NB, blk_body, 0)
      return carry
    lax.fori_loop(0, NCH, chunk_body, 0)
    plsc.subcore_barrier()

    pltpu.sync_copy(acc.at[pl.ds(sid * RPS, RPS)],
                    s_out.at[q, pl.ds(sid * RPS, RPS)])


_conv_kernel = functools.partial(
    pl.kernel,
    out_type=[
        jax.ShapeDtypeStruct((4, NP, L), jnp.float32),
        jax.ShapeDtypeStruct((G, 64), jnp.float32),
    ],
    mesh=_sc_mesh(),
    compiler_params=_SC_PARAMS,
    scratch_types=[
        pltpu.VMEM_SHARED((NP, L), jnp.float32),
        pltpu.VMEM((NB, BK), jnp.int32),
        pltpu.VMEM((NB, BK), jnp.int32),
        pltpu.VMEM((BK, L), jnp.float32),
        pltpu.VMEM((ZR, L), jnp.float32),
        pltpu.VMEM((G,), jnp.int32),
        pltpu.VMEM((G, 64), jnp.float32),
    ],
)(_conv_body)


def _tca_body(x_ref, deg_ref, w1, b1, w2, b2, w3, b3, wc1,
              x1_ref, g1_ref, dinv_ref):
  xb = x_ref[...]
  h = xb * w1[...] + b1[...]
  h = jnp.dot(h, w2[...], precision="highest") + b2[...]
  h = jnp.dot(h, w3[...], precision="highest") + b3[...]
  x1_ref[...] = h
  deg = deg_ref[0] + deg_ref[1] + 1.0
  dinv = lax.rsqrt(deg)
  dinv_ref[...] = dinv
  g1_ref[...] = jnp.dot(h, wc1[...], precision="highest") * dinv


def _full(shape):
  return pl.BlockSpec(shape, lambda i: tuple(0 for _ in shape))


def _tca(x, deg2, W1r, b1r, W2, b2r, W3, b3r, Wc1):
  return pl.pallas_call(
      _tca_body,
      grid=(NGRID,),
      in_specs=[
          pl.BlockSpec((NBLK, 1), lambda i: (i, 0)),
          pl.BlockSpec((NC, NBLK, 1), lambda i: (0, i, 0)),
          _full((1, 32)), _full((1, 32)),
          _full((32, 128)), _full((1, 128)),
          _full((128, 32)), _full((1, 32)),
          _full((32, 64)),
      ],
      out_specs=[
          pl.BlockSpec((NBLK, 32), lambda i: (i, 0)),
          pl.BlockSpec((NBLK, 64), lambda i: (i, 0)),
          pl.BlockSpec((NBLK, 1), lambda i: (i, 0)),
      ],
      out_shape=[
          jax.ShapeDtypeStruct((NP, 32), jnp.float32),
          jax.ShapeDtypeStruct((NP, 64), jnp.float32),
          jax.ShapeDtypeStruct((NP, 1), jnp.float32),
      ],
  )(x, deg2, W1r, b1r, W2, b2r, W3, b3r, Wc1)


def _tcb_body(s1_ref, g1_ref, dinv_ref, bc1, rv1_ref, batch_ref,
              wc2a, wc2b, x2_ref, g2_ref):
  s = s1_ref[...]
  scat = jnp.concatenate([s[0], s[1], s[2], s[3]], axis=1)
  dinv = dinv_ref[...]
  x2 = dinv * (scat + g1_ref[...]) + bc1[...]
  x2_ref[...] = x2
  oh = (batch_ref[...] == jnp.arange(G, dtype=jnp.int32)[None, :]
        ).astype(jnp.float32)
  rext = jnp.dot(oh, rv1_ref[...], precision="highest")
  h2 = (jnp.dot(jnp.maximum(x2, 0.0), wc2a[...], precision="highest")
        + jnp.dot(jnp.maximum(rext, 0.0), wc2b[...], precision="highest"))
  g2_ref[...] = h2 * dinv


def _tcb(s1, g1, dinv, bc1r, rv1, batch2, Wc2a, Wc2b):
  return pl.pallas_call(
      _tcb_body,
      grid=(NGRID,),
      in_specs=[
          pl.BlockSpec((4, NBLK, L), lambda i: (0, i, 0)),
          pl.BlockSpec((NBLK, 64), lambda i: (i, 0)),
          pl.BlockSpec((NBLK, 1), lambda i: (i, 0)),
          _full((1, 64)),
          _full((G, 32)),
          pl.BlockSpec((NBLK, 1), lambda i: (i, 0)),
          _full((64, 64)), _full((32, 64)),
      ],
      out_specs=[
          pl.BlockSpec((NBLK, 64), lambda i: (i, 0)),
          pl.BlockSpec((NBLK, 64), lambda i: (i, 0)),
      ],
      out_shape=[
          jax.ShapeDtypeStruct((NP, 64), jnp.float32),
          jax.ShapeDtypeStruct((NP, 64), jnp.float32),
      ],
  )(s1, g1, dinv, bc1r, rv1, batch2, Wc2a, Wc2b)


def _tcc_body(s2_ref, g2_ref, dinv_ref, bc2, rv2_ref, batch_ref,
              out_ref, seg_acc, cnt_acc):
  i = pl.program_id(0)
  s = s2_ref[...]
  scat = jnp.concatenate([s[0], s[1], s[2], s[3]], axis=1)
  x3 = jnp.maximum(dinv_ref[...] * (scat + g2_ref[...]) + bc2[...], 0.0)
  oh = (batch_ref[...] == jnp.arange(G, dtype=jnp.int32)[None, :]
        ).astype(jnp.float32)
  rext = jnp.dot(oh, rv2_ref[...], precision="highest")
  xc = jnp.concatenate([x3, rext], axis=1)
  part = lax.dot_general(oh, xc, (((0,), (0,)), ((), ())),
                         precision="highest")
  cntp = lax.dot_general(oh, jnp.ones((NBLK, 1), jnp.float32),
                         (((0,), (0,)), ((), ())), precision="highest")

  @pl.when(i == 0)
  def _init():
    seg_acc[...] = part
    cnt_acc[...] = cntp

  @pl.when(i > 0)
  def _accum():
    seg_acc[...] = seg_acc[...] + part
    cnt_acc[...] = cnt_acc[...] + cntp

  @pl.when(i == NGRID - 1)
  def _final():
    out_ref[...] = seg_acc[...] / jnp.maximum(cnt_acc[...], 1.0)


def _tcc(s2, g2, dinv, bc2r, rv2, batch2):
  return pl.pallas_call(
      _tcc_body,
      grid=(NGRID,),
      in_specs=[
          pl.BlockSpec((4, NBLK, L), lambda i: (0, i, 0)),
          pl.BlockSpec((NBLK, 64), lambda i: (i, 0)),
          pl.BlockSpec((NBLK, 1), lambda i: (i, 0)),
          _full((1, 64)),
          _full((G, 64)),
          pl.BlockSpec((NBLK, 1), lambda i: (i, 0)),
      ],
      out_specs=pl.BlockSpec((G, 2 * 64), lambda i: (0, 0)),
      out_shape=jax.ShapeDtypeStruct((G, 2 * 64), jnp.float32),
      scratch_shapes=[
          pltpu.VMEM((G, 2 * 64), jnp.float32),
          pltpu.VMEM((G, 1), jnp.float32),
      ],
  )(s2, g2, dinv, bc2r, rv2, batch2)


def kernel(x, edge_index, batch, root_index, W1, b1, W2, b2, W3, b3,
           Wc1, bc1, Wc2, bc2):
  x = x.astype(jnp.float32)
  src = edge_index[0]
  dst = edge_index[1]

  xp = jnp.pad(x, ((0, NP - N), (0, 0)))
  batchp = jnp.pad(batch, (0, NP - N), constant_values=G).reshape(NP, 1)
  src4q = (src * 4)[None, :] + jnp.arange(4, dtype=jnp.int32)[:, None]
  src4q = src4q.reshape(4, EROWS, BK)
  dst2 = dst.reshape(EROWS, BK)
  W1r, b1r = W1.reshape(1, 32), b1.reshape(1, 32)
  b2r, b3r = b2.reshape(1, 128), b3.reshape(1, 32)
  bc1r, bc2r = bc1.reshape(1, 64), bc2.reshape(1, 64)
  Wc2a, Wc2b = Wc2[:64], Wc2[64:]

  deg2, _ = _deg_kernel(dst)
  x1, g1, dinv = _tca(xp, deg2.reshape(NC, NP, 1),
                      W1r, b1r, W2, b2r, W3, b3r, Wc1)

  x1w = jnp.pad(x1, ((0, 0), (0, 32)))
  s1, rv1 = _conv_kernel(g1.reshape(4 * NP, L), src4q, dst2,
                         root_index, x1w)
  x2, g2 = _tcb(s1, g1, dinv, bc1r, rv1[:, :32], batchp, Wc2a, Wc2b)

  s2, rv2 = _conv_kernel(g2.reshape(4 * NP, L), src4q, dst2,
                         root_index, x2)
  return _tcc(s2, g2, dinv, bc2r, rv2, batchp)


# speedup vs baseline: 14.8517x; 1.7005x over previous
import functools

import jax
import jax.numpy as jnp
from jax import lax
from jax.experimental import pallas as pl
from jax.experimental.pallas import tpu as pltpu
from jax.experimental.pallas import tpu_sc as plsc

N = 100000
E = 1600000
G = 128
NC, NS, L = 2, 16, 16
NP = 100352

BK = 128
EP = 1605632
EROWS = EP // BK
RPT = EROWS // NS
NB = 28
NCH = RPT // NB
R = 7

RPS = NP // NS
ZR = 98

NW = NC * NS
EPT = E // NW
DCH = 2000
PW = 896

NBLK = 512
NGRID = NP // NBLK


def _sc_mesh():
  return plsc.VectorSubcoreMesh(
      core_axis_name="c", subcore_axis_name="s",
      num_cores=NC, num_subcores=NS)


def _deg_body(dst_hbm, out_hbm, part_hbm, hist_v, idx_v, piece_v, res_v):
  cid = lax.axis_index("c")
  sid = lax.axis_index("s")
  wid = sid * NC + cid

  def zero_body(i, carry):
    hist_v[pl.ds(i * L, L)] = jnp.zeros((L,), jnp.float32)
    return carry
  lax.fori_loop(0, NP // L, zero_body, 0)

  ones = jnp.ones((L,), jnp.float32)
  base = wid * EPT

  def chunk_body(j, carry):
    pltpu.sync_copy(dst_hbm.at[pl.ds(base + j * DCH, DCH)], idx_v)

    def scat_body(k, c2):
      idx = idx_v[pl.ds(k * L, L)]
      plsc.addupdate_scatter(hist_v, [idx], ones)
      return c2
    lax.fori_loop(0, DCH // L, scat_body, 0)
    return carry
  lax.fori_loop(0, EPT // DCH, chunk_body, 0)

  pltpu.sync_copy(hist_v, part_hbm.at[wid])
  plsc.subcore_barrier()

  colbase = sid * RPS
  for p in range(7):
    for r in range(NS):
      pltpu.sync_copy(
          part_hbm.at[r * NC + cid, pl.ds(colbase + p * PW, PW)],
          piece_v.at[r])

    def red_body(k, carry):
      sl = pl.ds(k * L, L)
      acc = piece_v[0, sl]
      for r in range(1, NS):
        acc = acc + piece_v[r, sl]
      res_v[sl] = acc
      return carry
    lax.fori_loop(0, PW // L, red_body, 0)
    pltpu.sync_copy(res_v, out_hbm.at[cid, pl.ds(colbase + p * PW, PW)])


_SC_PARAMS = pltpu.CompilerParams(
    use_tc_tiling_on_sc=False, needs_layout_passes=False)

_deg_kernel = functools.partial(
    pl.kernel,
    out_type=[
        jax.ShapeDtypeStruct((NC, NP), jnp.float32),
        jax.ShapeDtypeStruct((NW, NP), jnp.float32),
    ],
    mesh=_sc_mesh(),
    compiler_params=_SC_PARAMS,
    scratch_types=[
        pltpu.VMEM((NP,), jnp.float32),
        pltpu.VMEM((DCH,), jnp.int32),
        pltpu.VMEM((NS, PW), jnp.float32),
        pltpu.VMEM((PW,), jnp.float32),
    ],
)(_deg_body)


def _conv_body(g4_hbm, src4q_hbm, dst_hbm, ridx_hbm, xtab_hbm,
               s_out, rv_out, acc, srcq_v, dst_v, ring_v, zbuf_v,
               ridx_v, rrows_v, gsem, ssem):
  cid = lax.axis_index("c")
  sid = lax.axis_index("s")

  @pl.when(jnp.logical_and(cid == 0, sid == 0))
  def _root_gather():
    pltpu.sync_copy(ridx_hbm, ridx_v)
    for h in range(2):
      pltpu.sync_copy(xtab_hbm.at[ridx_v.at[pl.ds(h * 64, 64)]], rrows_v)
      pltpu.sync_copy(rrows_v, rv_out.at[pl.ds(h * 64, 64)])

  def zb_body(i, carry):
    zbuf_v[i, :] = jnp.zeros((L,), jnp.float32)
    return carry
  lax.fori_loop(0, ZR, zb_body, 0)

  for t in range(2):
    q = 2 * cid + t

    def z_body(z, carry):
      pltpu.sync_copy(zbuf_v, acc.at[pl.ds(sid * RPS + z * ZR, ZR)])
      return carry
    lax.fori_loop(0, RPS // ZR, z_body, 0)
    plsc.subcore_barrier()

    def chunk_body(j, carry):
      r0 = sid * RPT + j * NB
      pltpu.sync_copy(src4q_hbm.at[q, pl.ds(r0, NB)], srcq_v)
      pltpu.sync_copy(dst_hbm.at[pl.ds(r0, NB)], dst_v)
      for g in range(NB // R):
        gds = [
            pltpu.async_copy(g4_hbm.at[srcq_v.at[g * R + b]],
                             ring_v.at[b], gsem)
            for b in range(R)
        ]
        for d in gds:
          d.wait()
        sds = [
            pltpu.async_copy(ring_v.at[b], acc.at[dst_v.at[g * R + b]],
                             ssem, add=True)
            for b in range(R)
        ]
        for d in sds:
          d.wait()
      return carry
    lax.fori_loop(0, NCH, chunk_body, 0)
    plsc.subcore_barrier()

    pltpu.sync_copy(acc.at[pl.ds(sid * RPS, RPS)],
                    s_out.at[q, pl.ds(sid * RPS, RPS)])


_conv_kernel = functools.partial(
    pl.kernel,
    out_type=[
        jax.ShapeDtypeStruct((4, NP, L), jnp.float32),
        jax.ShapeDtypeStruct((G, 64), jnp.float32),
    ],
    mesh=_sc_mesh(),
    compiler_params=_SC_PARAMS,
    scratch_types=[
        pltpu.VMEM_SHARED((NP, L), jnp.float32),
        pltpu.VMEM((NB, BK), jnp.int32),
        pltpu.VMEM((NB, BK), jnp.int32),
        pltpu.VMEM((R, BK, L), jnp.float32),
        pltpu.VMEM((ZR, L), jnp.float32),
        pltpu.VMEM((G,), jnp.int32),
        pltpu.VMEM((64, 64), jnp.float32),
        pltpu.SemaphoreType.DMA,
        pltpu.SemaphoreType.DMA,
    ],
)(_conv_body)


def _tca_body(x_ref, deg_ref, w1, b1, w2, b2, w3, b3, wc1,
              x1_ref, g1_ref, dinv_ref):
  xb = x_ref[...]
  h = xb * w1[...] + b1[...]
  h = jnp.dot(h, w2[...], precision="highest") + b2[...]
  h = jnp.dot(h, w3[...], precision="highest") + b3[...]
  x1_ref[...] = h
  deg = deg_ref[0] + deg_ref[1] + 1.0
  dinv = lax.rsqrt(deg)
  dinv_ref[...] = dinv
  g1_ref[...] = jnp.dot(h, wc1[...], precision="highest") * dinv


def _full(shape):
  return pl.BlockSpec(shape, lambda i: tuple(0 for _ in shape))


def _tca(x, deg2, W1r, b1r, W2, b2r, W3, b3r, Wc1):
  return pl.pallas_call(
      _tca_body,
      grid=(NGRID,),
      in_specs=[
          pl.BlockSpec((NBLK, 1), lambda i: (i, 0)),
          pl.BlockSpec((NC, NBLK, 1), lambda i: (0, i, 0)),
          _full((1, 32)), _full((1, 32)),
          _full((32, 128)), _full((1, 128)),
          _full((128, 32)), _full((1, 32)),
          _full((32, 64)),
      ],
      out_specs=[
          pl.BlockSpec((NBLK, 32), lambda i: (i, 0)),
          pl.BlockSpec((NBLK, 64), lambda i: (i, 0)),
          pl.BlockSpec((NBLK, 1), lambda i: (i, 0)),
      ],
      out_shape=[
          jax.ShapeDtypeStruct((NP, 32), jnp.float32),
          jax.ShapeDtypeStruct((NP, 64), jnp.float32),
          jax.ShapeDtypeStruct((NP, 1), jnp.float32),
      ],
  )(x, deg2, W1r, b1r, W2, b2r, W3, b3r, Wc1)


def _tcb_body(s1_ref, g1_ref, dinv_ref, bc1, rv1_ref, batch_ref,
              wc2a, wc2b, x2_ref, g2_ref):
  s = s1_ref[...]
  scat = jnp.concatenate([s[0], s[1], s[2], s[3]], axis=1)
  dinv = dinv_ref[...]
  x2 = dinv * (scat + g1_ref[...]) + bc1[...]
  x2_ref[...] = x2
  oh = (batch_ref[...] == jnp.arange(G, dtype=jnp.int32)[None, :]
        ).astype(jnp.float32)
  rext = jnp.dot(oh, rv1_ref[...], precision="highest")
  h2 = (jnp.dot(jnp.maximum(x2, 0.0), wc2a[...], precision="highest")
        + jnp.dot(jnp.maximum(rext, 0.0), wc2b[...], precision="highest"))
  g2_ref[...] = h2 * dinv


def _tcb(s1, g1, dinv, bc1r, rv1, batch2, Wc2a, Wc2b):
  return pl.pallas_call(
      _tcb_body,
      grid=(NGRID,),
      in_specs=[
          pl.BlockSpec((4, NBLK, L), lambda i: (0, i, 0)),
          pl.BlockSpec((NBLK, 64), lambda i: (i, 0)),
          pl.BlockSpec((NBLK, 1), lambda i: (i, 0)),
          _full((1, 64)),
          _full((G, 32)),
          pl.BlockSpec((NBLK, 1), lambda i: (i, 0)),
          _full((64, 64)), _full((32, 64)),
      ],
      out_specs=[
          pl.BlockSpec((NBLK, 64), lambda i: (i, 0)),
          pl.BlockSpec((NBLK, 64), lambda i: (i, 0)),
      ],
      out_shape=[
          jax.ShapeDtypeStruct((NP, 64), jnp.float32),
          jax.ShapeDtypeStruct((NP, 64), jnp.float32),
      ],
  )(s1, g1, dinv, bc1r, rv1, batch2, Wc2a, Wc2b)


def _tcc_body(s2_ref, g2_ref, dinv_ref, bc2, rv2_ref, batch_ref,
              out_ref, seg_acc, cnt_acc):
  i = pl.program_id(0)
  s = s2_ref[...]
  scat = jnp.concatenate([s[0], s[1], s[2], s[3]], axis=1)
  x3 = jnp.maximum(dinv_ref[...] * (scat + g2_ref[...]) + bc2[...], 0.0)
  oh = (batch_ref[...] == jnp.arange(G, dtype=jnp.int32)[None, :]
        ).astype(jnp.float32)
  rext = jnp.dot(oh, rv2_ref[...], precision="highest")
  xc = jnp.concatenate([x3, rext], axis=1)
  part = lax.dot_general(oh, xc, (((0,), (0,)), ((), ())),
                         precision="highest")
  cntp = lax.dot_general(oh, jnp.ones((NBLK, 1), jnp.float32),
                         (((0,), (0,)), ((), ())), precision="highest")

  @pl.when(i == 0)
  def _init():
    seg_acc[...] = part
    cnt_acc[...] = cntp

  @pl.when(i > 0)
  def _accum():
    seg_acc[...] = seg_acc[...] + part
    cnt_acc[...] = cnt_acc[...] + cntp

  @pl.when(i == NGRID - 1)
  def _final():
    out_ref[...] = seg_acc[...] / jnp.maximum(cnt_acc[...], 1.0)


def _tcc(s2, g2, dinv, bc2r, rv2, batch2):
  return pl.pallas_call(
      _tcc_body,
      grid=(NGRID,),
      in_specs=[
          pl.BlockSpec((4, NBLK, L), lambda i: (0, i, 0)),
          pl.BlockSpec((NBLK, 64), lambda i: (i, 0)),
          pl.BlockSpec((NBLK, 1), lambda i: (i, 0)),
          _full((1, 64)),
          _full((G, 64)),
          pl.BlockSpec((NBLK, 1), lambda i: (i, 0)),
      ],
      out_specs=pl.BlockSpec((G, 2 * 64), lambda i: (0, 0)),
      out_shape=jax.ShapeDtypeStruct((G, 2 * 64), jnp.float32),
      scratch_shapes=[
          pltpu.VMEM((G, 2 * 64), jnp.float32),
          pltpu.VMEM((G, 1), jnp.float32),
      ],
  )(s2, g2, dinv, bc2r, rv2, batch2)


def kernel(x, edge_index, batch, root_index, W1, b1, W2, b2, W3, b3,
           Wc1, bc1, Wc2, bc2):
  x = x.astype(jnp.float32)
  src = edge_index[0]
  dst = edge_index[1]

  xp = jnp.pad(x, ((0, NP - N), (0, 0)))
  batchp = jnp.pad(batch, (0, NP - N), constant_values=G).reshape(NP, 1)
  srcp = jnp.pad(src, (0, EP - E))
  dstp = jnp.pad(dst, (0, EP - E), constant_values=N)
  src4q = (srcp * 4)[None, :] + jnp.arange(4, dtype=jnp.int32)[:, None]
  src4q = src4q.reshape(4, EROWS, BK)
  dst2 = dstp.reshape(EROWS, BK)
  W1r, b1r = W1.reshape(1, 32), b1.reshape(1, 32)
  b2r, b3r = b2.reshape(1, 128), b3.reshape(1, 32)
  bc1r, bc2r = bc1.reshape(1, 64), bc2.reshape(1, 64)
  Wc2a, Wc2b = Wc2[:64], Wc2[64:]

  deg2, _ = _deg_kernel(dst)
  x1, g1, dinv = _tca(xp, deg2.reshape(NC, NP, 1),
                      W1r, b1r, W2, b2r, W3, b3r, Wc1)

  x1w = jnp.pad(x1, ((0, 0), (0, 32)))
  s1, rv1 = _conv_kernel(g1.reshape(4 * NP, L), src4q, dst2,
                         root_index, x1w)
  x2, g2 = _tcb(s1, g1, dinv, bc1r, rv1[:, :32], batchp, Wc2a, Wc2b)

  s2, rv2 = _conv_kernel(g2.reshape(4 * NP, L), src4q, dst2,
                         root_index, x2)
  return _tcc(s2, g2, dinv, bc2r, rv2, batchp)


# speedup vs baseline: 16.4246x; 1.1059x over previous
import functools

import jax
import jax.numpy as jnp
from jax import lax
from jax.experimental import pallas as pl
from jax.experimental.pallas import tpu as pltpu
from jax.experimental.pallas import tpu_sc as plsc

N = 100000
E = 1600000
G = 128
NC, NS, L = 2, 16, 16
NP = 100352

BK = 128
EP = 1605632
EROWS = EP // BK
RPT = EROWS // NS
NB = 28
NCH = RPT // NB
R = 7

RPS = NP // NS
ZR = 98

NW = NC * NS
EPT = E // NW
DCH = 2000
PW = 896

NBLK = 2048
NGRID = NP // NBLK


def _sc_mesh():
  return plsc.VectorSubcoreMesh(
      core_axis_name="c", subcore_axis_name="s",
      num_cores=NC, num_subcores=NS)


def _deg_body(dst_hbm, out_hbm, part_hbm, hist_v, idx_v, piece_v, res_v):
  cid = lax.axis_index("c")
  sid = lax.axis_index("s")
  wid = sid * NC + cid

  def zero_body(i, carry):
    hist_v[pl.ds(i * L, L)] = jnp.zeros((L,), jnp.float32)
    return carry
  lax.fori_loop(0, NP // L, zero_body, 0)

  ones = jnp.ones((L,), jnp.float32)
  base = wid * EPT

  def chunk_body(j, carry):
    pltpu.sync_copy(dst_hbm.at[pl.ds(base + j * DCH, DCH)], idx_v)

    def scat_body(k, c2):
      idx = idx_v[pl.ds(k * L, L)]
      plsc.addupdate_scatter(hist_v, [idx], ones)
      return c2
    lax.fori_loop(0, DCH // L, scat_body, 0)
    return carry
  lax.fori_loop(0, EPT // DCH, chunk_body, 0)

  pltpu.sync_copy(hist_v, part_hbm.at[wid])
  plsc.subcore_barrier()

  colbase = sid * RPS
  for p in range(7):
    for r in range(NS):
      pltpu.sync_copy(
          part_hbm.at[r * NC + cid, pl.ds(colbase + p * PW, PW)],
          piece_v.at[r])

    def red_body(k, carry):
      sl = pl.ds(k * L, L)
      acc = piece_v[0, sl]
      for r in range(1, NS):
        acc = acc + piece_v[r, sl]
      res_v[sl] = acc
      return carry
    lax.fori_loop(0, PW // L, red_body, 0)
    pltpu.sync_copy(res_v, out_hbm.at[cid, pl.ds(colbase + p * PW, PW)])


_SC_PARAMS = pltpu.CompilerParams(
    use_tc_tiling_on_sc=False, needs_layout_passes=False)

_deg_kernel = functools.partial(
    pl.kernel,
    out_type=[
        jax.ShapeDtypeStruct((NC, NP), jnp.float32),
        jax.ShapeDtypeStruct((NW, NP), jnp.float32),
    ],
    mesh=_sc_mesh(),
    compiler_params=_SC_PARAMS,
    scratch_types=[
        pltpu.VMEM((NP,), jnp.float32),
        pltpu.VMEM((DCH,), jnp.int32),
        pltpu.VMEM((NS, PW), jnp.float32),
        pltpu.VMEM((PW,), jnp.float32),
    ],
)(_deg_body)


def _conv_body(g4_hbm, src4q_hbm, dst_hbm, ridx_hbm, xtab_hbm,
               s_out, rv_out, acc, srcq_v, dst_v, ring_v, zbuf_v,
               ridx_v, rrows_v, gsem, ssem):
  cid = lax.axis_index("c")
  sid = lax.axis_index("s")

  @pl.when(jnp.logical_and(cid == 0, sid == 0))
  def _root_gather():
    pltpu.sync_copy(ridx_hbm, ridx_v)
    for h in range(2):
      pltpu.sync_copy(xtab_hbm.at[ridx_v.at[pl.ds(h * 64, 64)]], rrows_v)
      pltpu.sync_copy(rrows_v, rv_out.at[pl.ds(h * 64, 64)])

  def zb_body(i, carry):
    zbuf_v[i, :] = jnp.zeros((L,), jnp.float32)
    return carry
  lax.fori_loop(0, ZR, zb_body, 0)

  for t in range(2):
    q = 2 * cid + t

    def z_body(z, carry):
      pltpu.sync_copy(zbuf_v, acc.at[pl.ds(sid * RPS + z * ZR, ZR)])
      return carry
    lax.fori_loop(0, RPS // ZR, z_body, 0)
    plsc.subcore_barrier()

    def chunk_body(j, carry):
      r0 = sid * RPT + j * NB
      pltpu.sync_copy(src4q_hbm.at[q, pl.ds(r0, NB)], srcq_v)
      pltpu.sync_copy(dst_hbm.at[pl.ds(r0, NB)], dst_v)
      for g in range(NB // R):
        gds = [
            pltpu.async_copy(g4_hbm.at[srcq_v.at[g * R + b]],
                             ring_v.at[b], gsem)
            for b in range(R)
        ]
        for d in gds:
          d.wait()
        sds = [
            pltpu.async_copy(ring_v.at[b], acc.at[dst_v.at[g * R + b]],
                             ssem, add=True)
            for b in range(R)
        ]
        for d in sds:
          d.wait()
      return carry
    lax.fori_loop(0, NCH, chunk_body, 0)
    plsc.subcore_barrier()

    pltpu.sync_copy(acc.at[pl.ds(sid * RPS, RPS)],
                    s_out.at[pl.ds(sid * RPS, RPS), pl.ds(q * L, L)])


_conv_kernel = functools.partial(
    pl.kernel,
    out_type=[
        jax.ShapeDtypeStruct((NP, 4 * L), jnp.float32),
        jax.ShapeDtypeStruct((G, 64), jnp.float32),
    ],
    mesh=_sc_mesh(),
    compiler_params=_SC_PARAMS,
    scratch_types=[
        pltpu.VMEM_SHARED((NP, L), jnp.float32),
        pltpu.VMEM((NB, BK), jnp.int32),
        pltpu.VMEM((NB, BK), jnp.int32),
        pltpu.VMEM((R, BK, L), jnp.float32),
        pltpu.VMEM((ZR, L), jnp.float32),
        pltpu.VMEM((G,), jnp.int32),
        pltpu.VMEM((64, 64), jnp.float32),
        pltpu.SemaphoreType.DMA,
        pltpu.SemaphoreType.DMA,
    ],
)(_conv_body)


def _tca_body(x_ref, deg_ref, w1, b1, w2, b2, w3, b3, wc1,
              x1_ref, g1_ref, dinv_ref):
  xb = x_ref[...]
  h = xb * w1[...] + b1[...]
  h = jnp.dot(h, w2[...], precision="highest") + b2[...]
  h = jnp.dot(h, w3[...], precision="highest") + b3[...]
  x1_ref[...] = h
  deg = deg_ref[0] + deg_ref[1] + 1.0
  dinv = lax.rsqrt(deg)
  dinv_ref[...] = dinv
  g1_ref[...] = jnp.dot(h, wc1[...], precision="highest") * dinv


def _full(shape):
  return pl.BlockSpec(shape, lambda i: tuple(0 for _ in shape))


def _tca(x, deg2, W1r, b1r, W2, b2r, W3, b3r, Wc1):
  return pl.pallas_call(
      _tca_body,
      grid=(NGRID,),
      in_specs=[
          pl.BlockSpec((NBLK, 1), lambda i: (i, 0)),
          pl.BlockSpec((NC, NBLK, 1), lambda i: (0, i, 0)),
          _full((1, 32)), _full((1, 32)),
          _full((32, 128)), _full((1, 128)),
          _full((128, 32)), _full((1, 32)),
          _full((32, 64)),
      ],
      out_specs=[
          pl.BlockSpec((NBLK, 32), lambda i: (i, 0)),
          pl.BlockSpec((NBLK, 64), lambda i: (i, 0)),
          pl.BlockSpec((NBLK, 1), lambda i: (i, 0)),
      ],
      out_shape=[
          jax.ShapeDtypeStruct((NP, 32), jnp.float32),
          jax.ShapeDtypeStruct((NP, 64), jnp.float32),
          jax.ShapeDtypeStruct((NP, 1), jnp.float32),
      ],
  )(x, deg2, W1r, b1r, W2, b2r, W3, b3r, Wc1)


def _tcb_body(s1_ref, g1_ref, dinv_ref, bc1, rv1_ref, batch_ref,
              wc2a, wc2b, x2_ref, g2_ref):
  dinv = dinv_ref[...]
  x2 = dinv * (s1_ref[...] + g1_ref[...]) + bc1[...]
  x2_ref[...] = x2
  oh = (batch_ref[...] == jnp.arange(G, dtype=jnp.int32)[None, :]
        ).astype(jnp.float32)
  rext = jnp.dot(oh, rv1_ref[...], precision="highest")
  h2 = (jnp.dot(jnp.maximum(x2, 0.0), wc2a[...], precision="highest")
        + jnp.dot(jnp.maximum(rext, 0.0), wc2b[...], precision="highest"))
  g2_ref[...] = h2 * dinv


def _tcb(s1, g1, dinv, bc1r, rv1, batch2, Wc2a, Wc2b):
  return pl.pallas_call(
      _tcb_body,
      grid=(NGRID,),
      in_specs=[
          pl.BlockSpec((NBLK, 64), lambda i: (i, 0)),
          pl.BlockSpec((NBLK, 64), lambda i: (i, 0)),
          pl.BlockSpec((NBLK, 1), lambda i: (i, 0)),
          _full((1, 64)),
          _full((G, 32)),
          pl.BlockSpec((NBLK, 1), lambda i: (i, 0)),
          _full((64, 64)), _full((32, 64)),
      ],
      out_specs=[
          pl.BlockSpec((NBLK, 64), lambda i: (i, 0)),
          pl.BlockSpec((NBLK, 64), lambda i: (i, 0)),
      ],
      out_shape=[
          jax.ShapeDtypeStruct((NP, 64), jnp.float32),
          jax.ShapeDtypeStruct((NP, 64), jnp.float32),
      ],
  )(s1, g1, dinv, bc1r, rv1, batch2, Wc2a, Wc2b)


def _tcc_body(s2_ref, g2_ref, dinv_ref, bc2, rv2_ref, batch_ref,
              out_ref, seg_acc, cnt_acc):
  i = pl.program_id(0)
  x3 = jnp.maximum(dinv_ref[...] * (s2_ref[...] + g2_ref[...]) + bc2[...],
                   0.0)
  oh = (batch_ref[...] == jnp.arange(G, dtype=jnp.int32)[None, :]
        ).astype(jnp.float32)
  rext = jnp.dot(oh, rv2_ref[...], precision="highest")
  xc = jnp.concatenate([x3, rext], axis=1)
  part = lax.dot_general(oh, xc, (((0,), (0,)), ((), ())),
                         precision="highest")
  cntp = lax.dot_general(oh, jnp.ones((NBLK, 1), jnp.float32),
                         (((0,), (0,)), ((), ())), precision="highest")

  @pl.when(i == 0)
  def _init():
    seg_acc[...] = part
    cnt_acc[...] = cntp

  @pl.when(i > 0)
  def _accum():
    seg_acc[...] = seg_acc[...] + part
    cnt_acc[...] = cnt_acc[...] + cntp

  @pl.when(i == NGRID - 1)
  def _final():
    out_ref[...] = seg_acc[...] / jnp.maximum(cnt_acc[...], 1.0)


def _tcc(s2, g2, dinv, bc2r, rv2, batch2):
  return pl.pallas_call(
      _tcc_body,
      grid=(NGRID,),
      in_specs=[
          pl.BlockSpec((NBLK, 64), lambda i: (i, 0)),
          pl.BlockSpec((NBLK, 64), lambda i: (i, 0)),
          pl.BlockSpec((NBLK, 1), lambda i: (i, 0)),
          _full((1, 64)),
          _full((G, 64)),
          pl.BlockSpec((NBLK, 1), lambda i: (i, 0)),
      ],
      out_specs=pl.BlockSpec((G, 2 * 64), lambda i: (0, 0)),
      out_shape=jax.ShapeDtypeStruct((G, 2 * 64), jnp.float32),
      scratch_shapes=[
          pltpu.VMEM((G, 2 * 64), jnp.float32),
          pltpu.VMEM((G, 1), jnp.float32),
      ],
  )(s2, g2, dinv, bc2r, rv2, batch2)


def kernel(x, edge_index, batch, root_index, W1, b1, W2, b2, W3, b3,
           Wc1, bc1, Wc2, bc2):
  x = x.astype(jnp.float32)
  src = edge_index[0]
  dst = edge_index[1]

  xp = jnp.pad(x, ((0, NP - N), (0, 0)))
  batchp = jnp.pad(batch, (0, NP - N), constant_values=G).reshape(NP, 1)
  srcp = jnp.pad(src, (0, EP - E))
  dstp = jnp.pad(dst, (0, EP - E), constant_values=N)
  src4q = (srcp * 4)[None, :] + jnp.arange(4, dtype=jnp.int32)[:, None]
  src4q = src4q.reshape(4, EROWS, BK)
  dst2 = dstp.reshape(EROWS, BK)
  W1r, b1r = W1.reshape(1, 32), b1.reshape(1, 32)
  b2r, b3r = b2.reshape(1, 128), b3.reshape(1, 32)
  bc1r, bc2r = bc1.reshape(1, 64), bc2.reshape(1, 64)
  Wc2a, Wc2b = Wc2[:64], Wc2[64:]

  deg2, _ = _deg_kernel(dst)
  x1, g1, dinv = _tca(xp, deg2.reshape(NC, NP, 1),
                      W1r, b1r, W2, b2r, W3, b3r, Wc1)

  x1w = jnp.pad(x1, ((0, 0), (0, 32)))
  s1, rv1 = _conv_kernel(g1.reshape(4 * NP, L), src4q, dst2,
                         root_index, x1w)
  x2, g2 = _tcb(s1, g1, dinv, bc1r, rv1[:, :32], batchp, Wc2a, Wc2b)

  s2, rv2 = _conv_kernel(g2.reshape(4 * NP, L), src4q, dst2,
                         root_index, x2)
  return _tcc(s2, g2, dinv, bc2r, rv2, batchp)


# speedup vs baseline: 18.2953x; 1.1139x over previous
import functools

import jax
import jax.numpy as jnp
from jax import lax
from jax.experimental import pallas as pl
from jax.experimental.pallas import tpu as pltpu
from jax.experimental.pallas import tpu_sc as plsc

N = 100000
E = 1600000
G = 128
NC, NS, L = 2, 16, 16
NP = 100352

BK = 128
EP = 1605632
EROWS = EP // BK
RPT = EROWS // NS
NB = 28
NCH = RPT // NB
R = 7
LG = 3

RPS = NP // NS
ZR = 98

NW = NC * NS
EPT = E // NW
DCH = 2000
PW = 896

NBLK = 2048
NGRID = NP // NBLK


def _sc_mesh():
  return plsc.VectorSubcoreMesh(
      core_axis_name="c", subcore_axis_name="s",
      num_cores=NC, num_subcores=NS)


def _deg_body(dst_hbm, out_hbm, part_hbm, hist_v, idx_v, piece_v, res_v):
  cid = lax.axis_index("c")
  sid = lax.axis_index("s")
  wid = sid * NC + cid

  def zero_body(i, carry):
    hist_v[pl.ds(i * L, L)] = jnp.zeros((L,), jnp.float32)
    return carry
  lax.fori_loop(0, NP // L, zero_body, 0)

  ones = jnp.ones((L,), jnp.float32)
  base = wid * EPT

  def chunk_body(j, carry):
    pltpu.sync_copy(dst_hbm.at[pl.ds(base + j * DCH, DCH)], idx_v)

    def scat_body(k, c2):
      idx = idx_v[pl.ds(k * L, L)]
      plsc.addupdate_scatter(hist_v, [idx], ones)
      return c2
    lax.fori_loop(0, DCH // L, scat_body, 0)
    return carry
  lax.fori_loop(0, EPT // DCH, chunk_body, 0)

  pltpu.sync_copy(hist_v, part_hbm.at[wid])
  plsc.subcore_barrier()

  colbase = sid * RPS
  for p in range(7):
    for r in range(NS):
      pltpu.sync_copy(
          part_hbm.at[r * NC + cid, pl.ds(colbase + p * PW, PW)],
          piece_v.at[r])

    def red_body(k, carry):
      sl = pl.ds(k * L, L)
      acc = piece_v[0, sl]
      for r in range(1, NS):
        acc = acc + piece_v[r, sl]
      res_v[sl] = acc
      return carry
    lax.fori_loop(0, PW // L, red_body, 0)
    pltpu.sync_copy(res_v, out_hbm.at[cid, pl.ds(colbase + p * PW, PW)])


_SC_PARAMS = pltpu.CompilerParams(
    use_tc_tiling_on_sc=False, needs_layout_passes=False)

_deg_kernel = functools.partial(
    pl.kernel,
    out_type=[
        jax.ShapeDtypeStruct((NC, NP), jnp.float32),
        jax.ShapeDtypeStruct((NW, NP), jnp.float32),
    ],
    mesh=_sc_mesh(),
    compiler_params=_SC_PARAMS,
    scratch_types=[
        pltpu.VMEM((NP,), jnp.float32),
        pltpu.VMEM((DCH,), jnp.int32),
        pltpu.VMEM((NS, PW), jnp.float32),
        pltpu.VMEM((PW,), jnp.float32),
    ],
)(_deg_body)


def _conv_body(g4_hbm, src4q_hbm, dst_hbm, ridx_hbm, xtab_hbm,
               s_out, rv_out, acc, srcq_v, dst_v, ring_v, zbuf_v,
               ridx_v, rrows_v, gsem, ssem):
  cid = lax.axis_index("c")
  sid = lax.axis_index("s")

  @pl.when(jnp.logical_and(cid == 0, sid == 0))
  def _root_gather():
    pltpu.sync_copy(ridx_hbm, ridx_v)
    for h in range(2):
      pltpu.sync_copy(xtab_hbm.at[ridx_v.at[pl.ds(h * 64, 64)]], rrows_v)
      pltpu.sync_copy(rrows_v, rv_out.at[pl.ds(h * 64, 64)])

  def zb_body(i, carry):
    zbuf_v[i, :] = jnp.zeros((L,), jnp.float32)
    return carry
  lax.fori_loop(0, ZR, zb_body, 0)

  for t in range(2):
    q = 2 * cid + t

    def z_body(z, carry):
      pltpu.sync_copy(zbuf_v, acc.at[pl.ds(sid * RPS + z * ZR, ZR)])
      return carry
    lax.fori_loop(0, RPS // ZR, z_body, 0)
    plsc.subcore_barrier()

    def chunk_body(j, carry):
      r0 = sid * RPT + j * NB
      pltpu.sync_copy(src4q_hbm.at[q, pl.ds(r0, NB)], srcq_v)
      pltpu.sync_copy(dst_hbm.at[pl.ds(r0, NB)], dst_v)
      gds = [None] * NB
      sds = [None] * NB
      for jj in range(NB + LG):
        if jj < NB:
          slot = jj % R
          if jj >= R:
            sds[jj - R].wait()
          gds[jj] = pltpu.async_copy(g4_hbm.at[srcq_v.at[jj]],
                                     ring_v.at[slot], gsem.at[slot])
        if jj >= LG:
          k = jj - LG
          gds[k].wait()
          sds[k] = pltpu.async_copy(ring_v.at[k % R],
                                    acc.at[dst_v.at[k]],
                                    ssem.at[k % R], add=True)
      for k in range(NB - R, NB):
        sds[k].wait()
      return carry
    lax.fori_loop(0, NCH, chunk_body, 0)
    plsc.subcore_barrier()

    pltpu.sync_copy(acc.at[pl.ds(sid * RPS, RPS)],
                    s_out.at[pl.ds(sid * RPS, RPS), pl.ds(q * L, L)])


_conv_kernel = functools.partial(
    pl.kernel,
    out_type=[
        jax.ShapeDtypeStruct((NP, 4 * L), jnp.float32),
        jax.ShapeDtypeStruct((G, 64), jnp.float32),
    ],
    mesh=_sc_mesh(),
    compiler_params=_SC_PARAMS,
    scratch_types=[
        pltpu.VMEM_SHARED((NP, L), jnp.float32),
        pltpu.VMEM((NB, BK), jnp.int32),
        pltpu.VMEM((NB, BK), jnp.int32),
        pltpu.VMEM((R, BK, L), jnp.float32),
        pltpu.VMEM((ZR, L), jnp.float32),
        pltpu.VMEM((G,), jnp.int32),
        pltpu.VMEM((64, 64), jnp.float32),
        pltpu.SemaphoreType.DMA((R,)),
        pltpu.SemaphoreType.DMA((R,)),
    ],
)(_conv_body)


def _tca_body(x_ref, deg_ref, w1, b1, w2, b2, w3, b3, wc1,
              x1_ref, g1_ref, dinv_ref):
  xb = x_ref[...]
  h = xb * w1[...] + b1[...]
  h = jnp.dot(h, w2[...], precision="highest") + b2[...]
  h = jnp.dot(h, w3[...], precision="highest") + b3[...]
  x1_ref[...] = h
  deg = deg_ref[0] + deg_ref[1] + 1.0
  dinv = lax.rsqrt(deg)
  dinv_ref[...] = dinv
  g1_ref[...] = jnp.dot(h, wc1[...], precision="highest") * dinv


def _full(shape):
  return pl.BlockSpec(shape, lambda i: tuple(0 for _ in shape))


def _tca(x, deg2, W1r, b1r, W2, b2r, W3, b3r, Wc1):
  return pl.pallas_call(
      _tca_body,
      grid=(NGRID,),
      in_specs=[
          pl.BlockSpec((NBLK, 1), lambda i: (i, 0)),
          pl.BlockSpec((NC, NBLK, 1), lambda i: (0, i, 0)),
          _full((1, 32)), _full((1, 32)),
          _full((32, 128)), _full((1, 128)),
          _full((128, 32)), _full((1, 32)),
          _full((32, 64)),
      ],
      out_specs=[
          pl.BlockSpec((NBLK, 32), lambda i: (i, 0)),
          pl.BlockSpec((NBLK, 64), lambda i: (i, 0)),
          pl.BlockSpec((NBLK, 1), lambda i: (i, 0)),
      ],
      out_shape=[
          jax.ShapeDtypeStruct((NP, 32), jnp.float32),
          jax.ShapeDtypeStruct((NP, 64), jnp.float32),
          jax.ShapeDtypeStruct((NP, 1), jnp.float32),
      ],
  )(x, deg2, W1r, b1r, W2, b2r, W3, b3r, Wc1)


def _tcb_body(s1_ref, g1_ref, dinv_ref, bc1, rv1_ref, batch_ref,
              wc2a, wc2b, x2_ref, g2_ref):
  dinv = dinv_ref[...]
  x2 = dinv * (s1_ref[...] + g1_ref[...]) + bc1[...]
  x2_ref[...] = x2
  oh = (batch_ref[...] == jnp.arange(G, dtype=jnp.int32)[None, :]
        ).astype(jnp.float32)
  rext = jnp.dot(oh, rv1_ref[...], precision="highest")
  h2 = (jnp.dot(jnp.maximum(x2, 0.0), wc2a[...], precision="highest")
        + jnp.dot(jnp.maximum(rext, 0.0), wc2b[...], precision="highest"))
  g2_ref[...] = h2 * dinv


def _tcb(s1, g1, dinv, bc1r, rv1, batch2, Wc2a, Wc2b):
  return pl.pallas_call(
      _tcb_body,
      grid=(NGRID,),
      in_specs=[
          pl.BlockSpec((NBLK, 64), lambda i: (i, 0)),
          pl.BlockSpec((NBLK, 64), lambda i: (i, 0)),
          pl.BlockSpec((NBLK, 1), lambda i: (i, 0)),
          _full((1, 64)),
          _full((G, 32)),
          pl.BlockSpec((NBLK, 1), lambda i: (i, 0)),
          _full((64, 64)), _full((32, 64)),
      ],
      out_specs=[
          pl.BlockSpec((NBLK, 64), lambda i: (i, 0)),
          pl.BlockSpec((NBLK, 64), lambda i: (i, 0)),
      ],
      out_shape=[
          jax.ShapeDtypeStruct((NP, 64), jnp.float32),
          jax.ShapeDtypeStruct((NP, 64), jnp.float32),
      ],
  )(s1, g1, dinv, bc1r, rv1, batch2, Wc2a, Wc2b)


def _tcc_body(s2_ref, g2_ref, dinv_ref, bc2, rv2_ref, batch_ref,
              out_ref, seg_acc, cnt_acc):
  i = pl.program_id(0)
  x3 = jnp.maximum(dinv_ref[...] * (s2_ref[...] + g2_ref[...]) + bc2[...],
                   0.0)
  oh = (batch_ref[...] == jnp.arange(G, dtype=jnp.int32)[None, :]
        ).astype(jnp.float32)
  rext = jnp.dot(oh, rv2_ref[...], precision="highest")
  xc = jnp.concatenate([x3, rext], axis=1)
  part = lax.dot_general(oh, xc, (((0,), (0,)), ((), ())),
                         precision="highest")
  cntp = lax.dot_general(oh, jnp.ones((NBLK, 1), jnp.float32),
                         (((0,), (0,)), ((), ())), precision="highest")

  @pl.when(i == 0)
  def _init():
    seg_acc[...] = part
    cnt_acc[...] = cntp

  @pl.when(i > 0)
  def _accum():
    seg_acc[...] = seg_acc[...] + part
    cnt_acc[...] = cnt_acc[...] + cntp

  @pl.when(i == NGRID - 1)
  def _final():
    out_ref[...] = seg_acc[...] / jnp.maximum(cnt_acc[...], 1.0)


def _tcc(s2, g2, dinv, bc2r, rv2, batch2):
  return pl.pallas_call(
      _tcc_body,
      grid=(NGRID,),
      in_specs=[
          pl.BlockSpec((NBLK, 64), lambda i: (i, 0)),
          pl.BlockSpec((NBLK, 64), lambda i: (i, 0)),
          pl.BlockSpec((NBLK, 1), lambda i: (i, 0)),
          _full((1, 64)),
          _full((G, 64)),
          pl.BlockSpec((NBLK, 1), lambda i: (i, 0)),
      ],
      out_specs=pl.BlockSpec((G, 2 * 64), lambda i: (0, 0)),
      out_shape=jax.ShapeDtypeStruct((G, 2 * 64), jnp.float32),
      scratch_shapes=[
          pltpu.VMEM((G, 2 * 64), jnp.float32),
          pltpu.VMEM((G, 1), jnp.float32),
      ],
  )(s2, g2, dinv, bc2r, rv2, batch2)


def kernel(x, edge_index, batch, root_index, W1, b1, W2, b2, W3, b3,
           Wc1, bc1, Wc2, bc2):
  x = x.astype(jnp.float32)
  src = edge_index[0]
  dst = edge_index[1]

  xp = jnp.pad(x, ((0, NP - N), (0, 0)))
  batchp = jnp.pad(batch, (0, NP - N), constant_values=G).reshape(NP, 1)
  srcp = jnp.pad(src, (0, EP - E))
  dstp = jnp.pad(dst, (0, EP - E), constant_values=N)
  src4q = (srcp * 4)[None, :] + jnp.arange(4, dtype=jnp.int32)[:, None]
  src4q = src4q.reshape(4, EROWS, BK)
  dst2 = dstp.reshape(EROWS, BK)
  W1r, b1r = W1.reshape(1, 32), b1.reshape(1, 32)
  b2r, b3r = b2.reshape(1, 128), b3.reshape(1, 32)
  bc1r, bc2r = bc1.reshape(1, 64), bc2.reshape(1, 64)
  Wc2a, Wc2b = Wc2[:64], Wc2[64:]

  deg2, _ = _deg_kernel(dst)
  x1, g1, dinv = _tca(xp, deg2.reshape(NC, NP, 1),
                      W1r, b1r, W2, b2r, W3, b3r, Wc1)

  x1w = jnp.pad(x1, ((0, 0), (0, 32)))
  s1, rv1 = _conv_kernel(g1.reshape(4 * NP, L), src4q, dst2,
                         root_index, x1w)
  x2, g2 = _tcb(s1, g1, dinv, bc1r, rv1[:, :32], batchp, Wc2a, Wc2b)

  s2, rv2 = _conv_kernel(g2.reshape(4 * NP, L), src4q, dst2,
                         root_index, x2)
  return _tcc(s2, g2, dinv, bc2r, rv2, batchp)


# speedup vs baseline: 18.6975x; 1.0220x over previous
import functools

import jax
import jax.numpy as jnp
from jax import lax
from jax.experimental import pallas as pl
from jax.experimental.pallas import tpu as pltpu
from jax.experimental.pallas import tpu_sc as plsc

N = 100000
E = 1600000
G = 128
NC, NS, L = 2, 16, 16
NP = 100352

BK = 128
EP = 1605632
EROWS = EP // BK
RPT = EROWS // NS
NB = 14
NCH = RPT // NB
R = 12
LG = 8

RPS = NP // NS
ZR = 49

NW = NC * NS
EPT = E // NW
DCH = 2000
PW = 896

NBLK = 2048
NGRID = NP // NBLK


def _sc_mesh():
  return plsc.VectorSubcoreMesh(
      core_axis_name="c", subcore_axis_name="s",
      num_cores=NC, num_subcores=NS)


def _deg_body(dst_hbm, out_hbm, part_hbm, hist_v, idx_v, piece_v, res_v):
  cid = lax.axis_index("c")
  sid = lax.axis_index("s")
  wid = sid * NC + cid

  def zero_body(i, carry):
    hist_v[pl.ds(i * L, L)] = jnp.zeros((L,), jnp.float32)
    return carry
  lax.fori_loop(0, NP // L, zero_body, 0)

  ones = jnp.ones((L,), jnp.float32)
  base = wid * EPT

  def chunk_body(j, carry):
    pltpu.sync_copy(dst_hbm.at[pl.ds(base + j * DCH, DCH)], idx_v)

    def scat_body(k, c2):
      idx = idx_v[pl.ds(k * L, L)]
      plsc.addupdate_scatter(hist_v, [idx], ones)
      return c2
    lax.fori_loop(0, DCH // L, scat_body, 0)
    return carry
  lax.fori_loop(0, EPT // DCH, chunk_body, 0)

  pltpu.sync_copy(hist_v, part_hbm.at[wid])
  plsc.subcore_barrier()

  colbase = sid * RPS
  for p in range(7):
    for r in range(NS):
      pltpu.sync_copy(
          part_hbm.at[r * NC + cid, pl.ds(colbase + p * PW, PW)],
          piece_v.at[r])

    def red_body(k, carry):
      sl = pl.ds(k * L, L)
      acc = piece_v[0, sl]
      for r in range(1, NS):
        acc = acc + piece_v[r, sl]
      res_v[sl] = acc
      return carry
    lax.fori_loop(0, PW // L, red_body, 0)
    pltpu.sync_copy(res_v, out_hbm.at[cid, pl.ds(colbase + p * PW, PW)])


_SC_PARAMS = pltpu.CompilerParams(
    use_tc_tiling_on_sc=False, needs_layout_passes=False)

_deg_kernel = functools.partial(
    pl.kernel,
    out_type=[
        jax.ShapeDtypeStruct((NC, NP), jnp.float32),
        jax.ShapeDtypeStruct((NW, NP), jnp.float32),
    ],
    mesh=_sc_mesh(),
    compiler_params=_SC_PARAMS,
    scratch_types=[
        pltpu.VMEM((NP,), jnp.float32),
        pltpu.VMEM((DCH,), jnp.int32),
        pltpu.VMEM((NS, PW), jnp.float32),
        pltpu.VMEM((PW,), jnp.float32),
    ],
)(_deg_body)


def _conv_body(g4_hbm, src4q_hbm, dst_hbm, ridx_hbm, xtab_hbm,
               s_out, rv_out, acc, srcq_v, dst_v, ring_v, zbuf_v,
               ridx_v, rrows_v, gsem, ssem):
  cid = lax.axis_index("c")
  sid = lax.axis_index("s")

  @pl.when(jnp.logical_and(cid == 0, sid == 0))
  def _root_gather():
    pltpu.sync_copy(ridx_hbm, ridx_v)
    for h in range(8):
      pltpu.sync_copy(xtab_hbm.at[ridx_v.at[pl.ds(h * 16, 16)]], rrows_v)
      pltpu.sync_copy(rrows_v, rv_out.at[pl.ds(h * 16, 16)])

  def zb_body(i, carry):
    zbuf_v[i, :] = jnp.zeros((L,), jnp.float32)
    return carry
  lax.fori_loop(0, ZR, zb_body, 0)

  for t in range(2):
    q = 2 * cid + t

    def z_body(z, carry):
      pltpu.sync_copy(zbuf_v, acc.at[pl.ds(sid * RPS + z * ZR, ZR)])
      return carry
    lax.fori_loop(0, RPS // ZR, z_body, 0)
    plsc.subcore_barrier()

    def chunk_body(j, carry):
      r0 = sid * RPT + j * NB
      pltpu.sync_copy(src4q_hbm.at[q, pl.ds(r0, NB)], srcq_v)
      pltpu.sync_copy(dst_hbm.at[pl.ds(r0, NB)], dst_v)
      gds = [None] * NB
      sds = [None] * NB
      for jj in range(NB + LG):
        if jj < NB:
          slot = jj % R
          if jj >= R:
            sds[jj - R].wait()
          gds[jj] = pltpu.async_copy(g4_hbm.at[srcq_v.at[jj]],
                                     ring_v.at[slot], gsem.at[slot])
        if jj >= LG:
          k = jj - LG
          gds[k].wait()
          sds[k] = pltpu.async_copy(ring_v.at[k % R],
                                    acc.at[dst_v.at[k]],
                                    ssem.at[k % R], add=True)
      for k in range(NB - R, NB):
        sds[k].wait()
      return carry
    lax.fori_loop(0, NCH, chunk_body, 0)
    plsc.subcore_barrier()

    pltpu.sync_copy(acc.at[pl.ds(sid * RPS, RPS)],
                    s_out.at[pl.ds(sid * RPS, RPS), pl.ds(q * L, L)])


_conv_kernel = functools.partial(
    pl.kernel,
    out_type=[
        jax.ShapeDtypeStruct((NP, 4 * L), jnp.float32),
        jax.ShapeDtypeStruct((G, 64), jnp.float32),
    ],
    mesh=_sc_mesh(),
    compiler_params=_SC_PARAMS,
    scratch_types=[
        pltpu.VMEM_SHARED((NP, L), jnp.float32),
        pltpu.VMEM((NB, BK), jnp.int32),
        pltpu.VMEM((NB, BK), jnp.int32),
        pltpu.VMEM((R, BK, L), jnp.float32),
        pltpu.VMEM((ZR, L), jnp.float32),
        pltpu.VMEM((G,), jnp.int32),
        pltpu.VMEM((16, 64), jnp.float32),
        pltpu.SemaphoreType.DMA((R,)),
        pltpu.SemaphoreType.DMA((R,)),
    ],
)(_conv_body)


def _tca_body(x_ref, deg_ref, w1, b1, w2, b2, w3, b3, wc1,
              x1_ref, g1_ref, dinv_ref):
  xb = x_ref[...]
  h = xb * w1[...] + b1[...]
  h = jnp.dot(h, w2[...], precision="highest") + b2[...]
  h = jnp.dot(h, w3[...], precision="highest") + b3[...]
  x1_ref[...] = h
  deg = deg_ref[0] + deg_ref[1] + 1.0
  dinv = lax.rsqrt(deg)
  dinv_ref[...] = dinv
  g1_ref[...] = jnp.dot(h, wc1[...], precision="highest") * dinv


def _full(shape):
  return pl.BlockSpec(shape, lambda i: tuple(0 for _ in shape))


def _tca(x, deg2, W1r, b1r, W2, b2r, W3, b3r, Wc1):
  return pl.pallas_call(
      _tca_body,
      grid=(NGRID,),
      in_specs=[
          pl.BlockSpec((NBLK, 1), lambda i: (i, 0)),
          pl.BlockSpec((NC, NBLK, 1), lambda i: (0, i, 0)),
          _full((1, 32)), _full((1, 32)),
          _full((32, 128)), _full((1, 128)),
          _full((128, 32)), _full((1, 32)),
          _full((32, 64)),
      ],
      out_specs=[
          pl.BlockSpec((NBLK, 32), lambda i: (i, 0)),
          pl.BlockSpec((NBLK, 64), lambda i: (i, 0)),
          pl.BlockSpec((NBLK, 1), lambda i: (i, 0)),
      ],
      out_shape=[
          jax.ShapeDtypeStruct((NP, 32), jnp.float32),
          jax.ShapeDtypeStruct((NP, 64), jnp.float32),
          jax.ShapeDtypeStruct((NP, 1), jnp.float32),
      ],
  )(x, deg2, W1r, b1r, W2, b2r, W3, b3r, Wc1)


def _tcb_body(s1_ref, g1_ref, dinv_ref, bc1, rv1_ref, batch_ref,
              wc2a, wc2b, x2_ref, g2_ref):
  dinv = dinv_ref[...]
  x2 = dinv * (s1_ref[...] + g1_ref[...]) + bc1[...]
  x2_ref[...] = x2
  oh = (batch_ref[...] == jnp.arange(G, dtype=jnp.int32)[None, :]
        ).astype(jnp.float32)
  rext = jnp.dot(oh, rv1_ref[...], precision="highest")
  h2 = (jnp.dot(jnp.maximum(x2, 0.0), wc2a[...], precision="highest")
        + jnp.dot(jnp.maximum(rext, 0.0), wc2b[...], precision="highest"))
  g2_ref[...] = h2 * dinv


def _tcb(s1, g1, dinv, bc1r, rv1, batch2, Wc2a, Wc2b):
  return pl.pallas_call(
      _tcb_body,
      grid=(NGRID,),
      in_specs=[
          pl.BlockSpec((NBLK, 64), lambda i: (i, 0)),
          pl.BlockSpec((NBLK, 64), lambda i: (i, 0)),
          pl.BlockSpec((NBLK, 1), lambda i: (i, 0)),
          _full((1, 64)),
          _full((G, 32)),
          pl.BlockSpec((NBLK, 1), lambda i: (i, 0)),
          _full((64, 64)), _full((32, 64)),
      ],
      out_specs=[
          pl.BlockSpec((NBLK, 64), lambda i: (i, 0)),
          pl.BlockSpec((NBLK, 64), lambda i: (i, 0)),
      ],
      out_shape=[
          jax.ShapeDtypeStruct((NP, 64), jnp.float32),
          jax.ShapeDtypeStruct((NP, 64), jnp.float32),
      ],
  )(s1, g1, dinv, bc1r, rv1, batch2, Wc2a, Wc2b)


def _tcc_body(s2_ref, g2_ref, dinv_ref, bc2, rv2_ref, batch_ref,
              out_ref, seg_acc, cnt_acc):
  i = pl.program_id(0)
  x3 = jnp.maximum(dinv_ref[...] * (s2_ref[...] + g2_ref[...]) + bc2[...],
                   0.0)
  oh = (batch_ref[...] == jnp.arange(G, dtype=jnp.int32)[None, :]
        ).astype(jnp.float32)
  rext = jnp.dot(oh, rv2_ref[...], precision="highest")
  xc = jnp.concatenate([x3, rext], axis=1)
  part = lax.dot_general(oh, xc, (((0,), (0,)), ((), ())),
                         precision="highest")
  cntp = lax.dot_general(oh, jnp.ones((NBLK, 1), jnp.float32),
                         (((0,), (0,)), ((), ())), precision="highest")

  @pl.when(i == 0)
  def _init():
    seg_acc[...] = part
    cnt_acc[...] = cntp

  @pl.when(i > 0)
  def _accum():
    seg_acc[...] = seg_acc[...] + part
    cnt_acc[...] = cnt_acc[...] + cntp

  @pl.when(i == NGRID - 1)
  def _final():
    out_ref[...] = seg_acc[...] / jnp.maximum(cnt_acc[...], 1.0)


def _tcc(s2, g2, dinv, bc2r, rv2, batch2):
  return pl.pallas_call(
      _tcc_body,
      grid=(NGRID,),
      in_specs=[
          pl.BlockSpec((NBLK, 64), lambda i: (i, 0)),
          pl.BlockSpec((NBLK, 64), lambda i: (i, 0)),
          pl.BlockSpec((NBLK, 1), lambda i: (i, 0)),
          _full((1, 64)),
          _full((G, 64)),
          pl.BlockSpec((NBLK, 1), lambda i: (i, 0)),
      ],
      out_specs=pl.BlockSpec((G, 2 * 64), lambda i: (0, 0)),
      out_shape=jax.ShapeDtypeStruct((G, 2 * 64), jnp.float32),
      scratch_shapes=[
          pltpu.VMEM((G, 2 * 64), jnp.float32),
          pltpu.VMEM((G, 1), jnp.float32),
      ],
  )(s2, g2, dinv, bc2r, rv2, batch2)


def kernel(x, edge_index, batch, root_index, W1, b1, W2, b2, W3, b3,
           Wc1, bc1, Wc2, bc2):
  x = x.astype(jnp.float32)
  src = edge_index[0]
  dst = edge_index[1]

  xp = jnp.pad(x, ((0, NP - N), (0, 0)))
  batchp = jnp.pad(batch, (0, NP - N), constant_values=G).reshape(NP, 1)
  srcp = jnp.pad(src, (0, EP - E))
  dstp = jnp.pad(dst, (0, EP - E), constant_values=N)
  src4q = (srcp * 4)[None, :] + jnp.arange(4, dtype=jnp.int32)[:, None]
  src4q = src4q.reshape(4, EROWS, BK)
  dst2 = dstp.reshape(EROWS, BK)
  W1r, b1r = W1.reshape(1, 32), b1.reshape(1, 32)
  b2r, b3r = b2.reshape(1, 128), b3.reshape(1, 32)
  bc1r, bc2r = bc1.reshape(1, 64), bc2.reshape(1, 64)
  Wc2a, Wc2b = Wc2[:64], Wc2[64:]

  deg2, _ = _deg_kernel(dst)
  x1, g1, dinv = _tca(xp, deg2.reshape(NC, NP, 1),
                      W1r, b1r, W2, b2r, W3, b3r, Wc1)

  x1w = jnp.pad(x1, ((0, 0), (0, 32)))
  s1, rv1 = _conv_kernel(g1.reshape(4 * NP, L), src4q, dst2,
                         root_index, x1w)
  x2, g2 = _tcb(s1, g1, dinv, bc1r, rv1[:, :32], batchp, Wc2a, Wc2b)

  s2, rv2 = _conv_kernel(g2.reshape(4 * NP, L), src4q, dst2,
                         root_index, x2)
  return _tcc(s2, g2, dinv, bc2r, rv2, batchp)


# speedup vs baseline: 18.9935x; 1.0158x over previous
import functools

import jax
import jax.numpy as jnp
from jax import lax
from jax.experimental import pallas as pl
from jax.experimental.pallas import tpu as pltpu
from jax.experimental.pallas import tpu_sc as plsc

N = 100000
E = 1600000
G = 128
NC, NS, L = 2, 16, 16
NP = 100352

BK = 128
EP = 1605632
EROWS = EP // BK
RPT = EROWS // NS
NB = 14
NCH = RPT // NB
R = 12
LG = 8

RPS = NP // NS
ZR = 49

NW = NC * NS
EPT = E // NW
DCH = 2000
PW = 896

NBLK = 2048
NGRID = NP // NBLK


def _sc_mesh():
  return plsc.VectorSubcoreMesh(
      core_axis_name="c", subcore_axis_name="s",
      num_cores=NC, num_subcores=NS)


def _deg_body(dst_hbm, out_hbm, part_hbm, hist_v, idx_v, piece_v, res_v):
  cid = lax.axis_index("c")
  sid = lax.axis_index("s")
  wid = sid * NC + cid

  def zero_body(i, carry):
    hist_v[pl.ds(i * L, L)] = jnp.zeros((L,), jnp.float32)
    return carry
  lax.fori_loop(0, NP // L, zero_body, 0)

  ones = jnp.ones((L,), jnp.float32)
  base = wid * EPT

  def chunk_body(j, carry):
    pltpu.sync_copy(dst_hbm.at[pl.ds(base + j * DCH, DCH)], idx_v)

    def scat_body(k, c2):
      idx = idx_v[pl.ds(k * L, L)]
      plsc.addupdate_scatter(hist_v, [idx], ones)
      return c2
    lax.fori_loop(0, DCH // L, scat_body, 0)
    return carry
  lax.fori_loop(0, EPT // DCH, chunk_body, 0)

  pltpu.sync_copy(hist_v, part_hbm.at[wid])
  plsc.subcore_barrier()

  colbase = sid * RPS
  for p in range(7):
    for r in range(NS):
      pltpu.sync_copy(
          part_hbm.at[r * NC + cid, pl.ds(colbase + p * PW, PW)],
          piece_v.at[r])

    def red_body(k, carry):
      sl = pl.ds(k * L, L)
      acc = piece_v[0, sl]
      for r in range(1, NS):
        acc = acc + piece_v[r, sl]
      res_v[sl] = acc
      return carry
    lax.fori_loop(0, PW // L, red_body, 0)
    pltpu.sync_copy(res_v, out_hbm.at[cid, pl.ds(colbase + p * PW, PW)])


_SC_PARAMS = pltpu.CompilerParams(
    use_tc_tiling_on_sc=False, needs_layout_passes=False)

_deg_kernel = functools.partial(
    pl.kernel,
    out_type=[
        jax.ShapeDtypeStruct((NC, NP), jnp.float32),
        jax.ShapeDtypeStruct((NW, NP), jnp.float32),
    ],
    mesh=_sc_mesh(),
    compiler_params=_SC_PARAMS,
    scratch_types=[
        pltpu.VMEM((NP,), jnp.float32),
        pltpu.VMEM((DCH,), jnp.int32),
        pltpu.VMEM((NS, PW), jnp.float32),
        pltpu.VMEM((PW,), jnp.float32),
    ],
)(_deg_body)


def _conv_body(g4_hbm, src4q_hbm, dst_hbm, ridx_hbm, xtab_hbm,
               s_out, rv_out, acc, srcq_v, dst_v, ring_v, zbuf_v,
               ridx_v, rrows_v, gsem, ssem):
  cid = lax.axis_index("c")
  sid = lax.axis_index("s")

  @pl.when(jnp.logical_and(cid == 0, sid == 0))
  def _root_gather():
    pltpu.sync_copy(ridx_hbm, ridx_v)
    for h in range(8):
      pltpu.sync_copy(xtab_hbm.at[ridx_v.at[pl.ds(h * 16, 16)]], rrows_v)
      pltpu.sync_copy(rrows_v, rv_out.at[pl.ds(h * 16, 16)])

  def zb_body(i, carry):
    zbuf_v[i, :] = jnp.zeros((L,), jnp.float32)
    return carry
  lax.fori_loop(0, ZR, zb_body, 0)

  for t in range(2):
    q = 2 * cid + t

    def z_body(z, carry):
      pltpu.sync_copy(zbuf_v, acc.at[pl.ds(sid * RPS + z * ZR, ZR)])
      return carry
    lax.fori_loop(0, RPS // ZR, z_body, 0)
    plsc.subcore_barrier()

    def chunk_body(j, carry):
      r0 = sid * RPT + j * NB
      pltpu.sync_copy(src4q_hbm.at[q, pl.ds(r0, NB)], srcq_v)
      pltpu.sync_copy(dst_hbm.at[pl.ds(r0, NB)], dst_v)
      gds = [None] * NB
      sds = [None] * NB
      for jj in range(NB + LG):
        if jj < NB:
          slot = jj % R
          if jj >= R:
            sds[jj - R].wait()
          gds[jj] = pltpu.async_copy(g4_hbm.at[srcq_v.at[jj]],
                                     ring_v.at[slot], gsem.at[slot])
        if jj >= LG:
          k = jj - LG
          gds[k].wait()
          sds[k] = pltpu.async_copy(ring_v.at[k % R],
                                    acc.at[dst_v.at[k]],
                                    ssem.at[k % R], add=True)
      for k in range(NB - R, NB):
        sds[k].wait()
      return carry
    lax.fori_loop(0, NCH, chunk_body, 0)
    plsc.subcore_barrier()

    pltpu.sync_copy(acc.at[pl.ds(sid * RPS, RPS)],
                    s_out.at[pl.ds(sid * RPS, RPS), pl.ds(q * L, L)])


_conv_kernel = functools.partial(
    pl.kernel,
    out_type=[
        jax.ShapeDtypeStruct((NP, 4 * L), jnp.float32),
        jax.ShapeDtypeStruct((G, 64), jnp.float32),
    ],
    mesh=_sc_mesh(),
    compiler_params=_SC_PARAMS,
    scratch_types=[
        pltpu.VMEM_SHARED((NP, L), jnp.float32),
        pltpu.VMEM((NB, BK), jnp.int32),
        pltpu.VMEM((NB, BK), jnp.int32),
        pltpu.VMEM((R, BK, L), jnp.float32),
        pltpu.VMEM((ZR, L), jnp.float32),
        pltpu.VMEM((G,), jnp.int32),
        pltpu.VMEM((16, 64), jnp.float32),
        pltpu.SemaphoreType.DMA((R,)),
        pltpu.SemaphoreType.DMA((R,)),
    ],
)(_conv_body)


def _tca1_body(x_ref, w1, b1, w2, b2, w3, b3, x1_ref):
  xb = x_ref[...]
  h = xb * w1[...] + b1[...]
  h = jnp.dot(h, w2[...], precision="highest") + b2[...]
  h = jnp.dot(h, w3[...], precision="highest") + b3[...]
  x1_ref[...] = h


def _tca2_body(x1_ref, deg_ref, wc1, g1_ref, dinv_ref):
  deg = deg_ref[0] + deg_ref[1] + 1.0
  dinv = lax.rsqrt(deg)
  dinv_ref[...] = dinv
  g1_ref[...] = jnp.dot(x1_ref[...], wc1[...], precision="highest") * dinv


def _full(shape):
  return pl.BlockSpec(shape, lambda i: tuple(0 for _ in shape))


def _tca1(x, W1r, b1r, W2, b2r, W3, b3r):
  return pl.pallas_call(
      _tca1_body,
      grid=(NGRID,),
      in_specs=[
          pl.BlockSpec((NBLK, 1), lambda i: (i, 0)),
          _full((1, 32)), _full((1, 32)),
          _full((32, 128)), _full((1, 128)),
          _full((128, 32)), _full((1, 32)),
      ],
      out_specs=pl.BlockSpec((NBLK, 32), lambda i: (i, 0)),
      out_shape=jax.ShapeDtypeStruct((NP, 32), jnp.float32),
  )(x, W1r, b1r, W2, b2r, W3, b3r)


def _tca2(x1, deg2, Wc1):
  return pl.pallas_call(
      _tca2_body,
      grid=(NGRID,),
      in_specs=[
          pl.BlockSpec((NBLK, 32), lambda i: (i, 0)),
          pl.BlockSpec((NC, NBLK, 1), lambda i: (0, i, 0)),
          _full((32, 64)),
      ],
      out_specs=[
          pl.BlockSpec((NBLK, 64), lambda i: (i, 0)),
          pl.BlockSpec((NBLK, 1), lambda i: (i, 0)),
      ],
      out_shape=[
          jax.ShapeDtypeStruct((NP, 64), jnp.float32),
          jax.ShapeDtypeStruct((NP, 1), jnp.float32),
      ],
  )(x1, deg2, Wc1)


def _tcb_body(s1_ref, g1_ref, dinv_ref, bc1, rv1_ref, batch_ref,
              wc2a, wc2b, x2_ref, g2_ref):
  dinv = dinv_ref[...]
  x2 = dinv * (s1_ref[...] + g1_ref[...]) + bc1[...]
  x2_ref[...] = x2
  oh = (batch_ref[...] == jnp.arange(G, dtype=jnp.int32)[None, :]
        ).astype(jnp.float32)
  rext = jnp.dot(oh, rv1_ref[...], precision="highest")
  h2 = (jnp.dot(jnp.maximum(x2, 0.0), wc2a[...], precision="highest")
        + jnp.dot(jnp.maximum(rext, 0.0), wc2b[...], precision="highest"))
  g2_ref[...] = h2 * dinv


def _tcb(s1, g1, dinv, bc1r, rv1, batch2, Wc2a, Wc2b):
  return pl.pallas_call(
      _tcb_body,
      grid=(NGRID,),
      in_specs=[
          pl.BlockSpec((NBLK, 64), lambda i: (i, 0)),
          pl.BlockSpec((NBLK, 64), lambda i: (i, 0)),
          pl.BlockSpec((NBLK, 1), lambda i: (i, 0)),
          _full((1, 64)),
          _full((G, 32)),
          pl.BlockSpec((NBLK, 1), lambda i: (i, 0)),
          _full((64, 64)), _full((32, 64)),
      ],
      out_specs=[
          pl.BlockSpec((NBLK, 64), lambda i: (i, 0)),
          pl.BlockSpec((NBLK, 64), lambda i: (i, 0)),
      ],
      out_shape=[
          jax.ShapeDtypeStruct((NP, 64), jnp.float32),
          jax.ShapeDtypeStruct((NP, 64), jnp.float32),
      ],
  )(s1, g1, dinv, bc1r, rv1, batch2, Wc2a, Wc2b)


def _tcc_body(s2_ref, g2_ref, dinv_ref, bc2, rv2_ref, batch_ref,
              out_ref, seg_acc, cnt_acc):
  i = pl.program_id(0)
  x3 = jnp.maximum(dinv_ref[...] * (s2_ref[...] + g2_ref[...]) + bc2[...],
                   0.0)
  oh = (batch_ref[...] == jnp.arange(G, dtype=jnp.int32)[None, :]
        ).astype(jnp.float32)
  rext = jnp.dot(oh, rv2_ref[...], precision="highest")
  xc = jnp.concatenate([x3, rext], axis=1)
  part = lax.dot_general(oh, xc, (((0,), (0,)), ((), ())),
                         precision="highest")
  cntp = lax.dot_general(oh, jnp.ones((NBLK, 1), jnp.float32),
                         (((0,), (0,)), ((), ())), precision="highest")

  @pl.when(i == 0)
  def _init():
    seg_acc[...] = part
    cnt_acc[...] = cntp

  @pl.when(i > 0)
  def _accum():
    seg_acc[...] = seg_acc[...] + part
    cnt_acc[...] = cnt_acc[...] + cntp

  @pl.when(i == NGRID - 1)
  def _final():
    out_ref[...] = seg_acc[...] / jnp.maximum(cnt_acc[...], 1.0)


def _tcc(s2, g2, dinv, bc2r, rv2, batch2):
  return pl.pallas_call(
      _tcc_body,
      grid=(NGRID,),
      in_specs=[
          pl.BlockSpec((NBLK, 64), lambda i: (i, 0)),
          pl.BlockSpec((NBLK, 64), lambda i: (i, 0)),
          pl.BlockSpec((NBLK, 1), lambda i: (i, 0)),
          _full((1, 64)),
          _full((G, 64)),
          pl.BlockSpec((NBLK, 1), lambda i: (i, 0)),
      ],
      out_specs=pl.BlockSpec((G, 2 * 64), lambda i: (0, 0)),
      out_shape=jax.ShapeDtypeStruct((G, 2 * 64), jnp.float32),
      scratch_shapes=[
          pltpu.VMEM((G, 2 * 64), jnp.float32),
          pltpu.VMEM((G, 1), jnp.float32),
      ],
  )(s2, g2, dinv, bc2r, rv2, batch2)


def kernel(x, edge_index, batch, root_index, W1, b1, W2, b2, W3, b3,
           Wc1, bc1, Wc2, bc2):
  x = x.astype(jnp.float32)
  src = edge_index[0]
  dst = edge_index[1]

  xp = jnp.pad(x, ((0, NP - N), (0, 0)))
  batchp = jnp.pad(batch, (0, NP - N), constant_values=G).reshape(NP, 1)
  srcp = jnp.pad(src, (0, EP - E))
  dstp = jnp.pad(dst, (0, EP - E), constant_values=N)
  src4q = (srcp * 4)[None, :] + jnp.arange(4, dtype=jnp.int32)[:, None]
  src4q = src4q.reshape(4, EROWS, BK)
  dst2 = dstp.reshape(EROWS, BK)
  W1r, b1r = W1.reshape(1, 32), b1.reshape(1, 32)
  b2r, b3r = b2.reshape(1, 128), b3.reshape(1, 32)
  bc1r, bc2r = bc1.reshape(1, 64), bc2.reshape(1, 64)
  Wc2a, Wc2b = Wc2[:64], Wc2[64:]

  deg2, _ = _deg_kernel(dst)
  x1 = _tca1(xp, W1r, b1r, W2, b2r, W3, b3r)
  g1, dinv = _tca2(x1, deg2.reshape(NC, NP, 1), Wc1)

  x1w = jnp.pad(x1, ((0, 0), (0, 32)))
  s1, rv1 = _conv_kernel(g1.reshape(4 * NP, L), src4q, dst2,
                         root_index, x1w)
  x2, g2 = _tcb(s1, g1, dinv, bc1r, rv1[:, :32], batchp, Wc2a, Wc2b)

  s2, rv2 = _conv_kernel(g2.reshape(4 * NP, L), src4q, dst2,
                         root_index, x2)
  return _tcc(s2, g2, dinv, bc2r, rv2, batchp)


# speedup vs baseline: 20.2559x; 1.0665x over previous
import functools

import jax
import jax.numpy as jnp
from jax import lax
from jax.experimental import pallas as pl
from jax.experimental.pallas import tpu as pltpu
from jax.experimental.pallas import tpu_sc as plsc

N = 100000
E = 1600000
G = 128
NC, NS, L = 2, 16, 16
NP = 100352

BK = 128
EP = 1605632
EROWS = EP // BK
RPT = EROWS // NS
NB = 28
NCH = RPT // NB
R = 10
LG = 7

RPS = NP // NS
ZR = 49

NW = NC * NS
EPT = E // NW
DCH = 2000
PW = 896

NBLK = 2048
NGRID = NP // NBLK


def _sc_mesh():
  return plsc.VectorSubcoreMesh(
      core_axis_name="c", subcore_axis_name="s",
      num_cores=NC, num_subcores=NS)


def _deg_body(dst_hbm, out_hbm, part_hbm, hist_v, idx_v, piece_v, res_v):
  cid = lax.axis_index("c")
  sid = lax.axis_index("s")
  wid = sid * NC + cid

  def zero_body(i, carry):
    hist_v[pl.ds(i * L, L)] = jnp.zeros((L,), jnp.float32)
    return carry
  lax.fori_loop(0, NP // L, zero_body, 0)

  ones = jnp.ones((L,), jnp.float32)
  base = wid * EPT

  def chunk_body(j, carry):
    pltpu.sync_copy(dst_hbm.at[pl.ds(base + j * DCH, DCH)], idx_v)

    def scat_body(k, c2):
      idx = idx_v[pl.ds(k * L, L)]
      plsc.addupdate_scatter(hist_v, [idx], ones)
      return c2
    lax.fori_loop(0, DCH // L, scat_body, 0)
    return carry
  lax.fori_loop(0, EPT // DCH, chunk_body, 0)

  pltpu.sync_copy(hist_v, part_hbm.at[wid])
  plsc.subcore_barrier()

  colbase = sid * RPS
  for p in range(7):
    for r in range(NS):
      pltpu.sync_copy(
          part_hbm.at[r * NC + cid, pl.ds(colbase + p * PW, PW)],
          piece_v.at[r])

    def red_body(k, carry):
      sl = pl.ds(k * L, L)
      acc = piece_v[0, sl]
      for r in range(1, NS):
        acc = acc + piece_v[r, sl]
      res_v[sl] = acc
      return carry
    lax.fori_loop(0, PW // L, red_body, 0)
    pltpu.sync_copy(res_v, out_hbm.at[cid, pl.ds(colbase + p * PW, PW)])


_SC_PARAMS = pltpu.CompilerParams(
    use_tc_tiling_on_sc=False, needs_layout_passes=False)

_deg_kernel = functools.partial(
    pl.kernel,
    out_type=[
        jax.ShapeDtypeStruct((NC, NP), jnp.float32),
        jax.ShapeDtypeStruct((NW, NP), jnp.float32),
    ],
    mesh=_sc_mesh(),
    compiler_params=_SC_PARAMS,
    scratch_types=[
        pltpu.VMEM((NP,), jnp.float32),
        pltpu.VMEM((DCH,), jnp.int32),
        pltpu.VMEM((NS, PW), jnp.float32),
        pltpu.VMEM((PW,), jnp.float32),
    ],
)(_deg_body)


def _conv_body(g4_hbm, src4q_hbm, dst_hbm, ridx_hbm, xtab_hbm,
               s_out, rv_out, acc, srcq_v, dst_v, ring_v, zbuf_v,
               ridx_v, rrows_v, gsem, ssem):
  cid = lax.axis_index("c")
  sid = lax.axis_index("s")

  @pl.when(jnp.logical_and(cid == 0, sid == 0))
  def _root_gather():
    pltpu.sync_copy(ridx_hbm, ridx_v)
    for h in range(8):
      pltpu.sync_copy(xtab_hbm.at[ridx_v.at[pl.ds(h * 16, 16)]], rrows_v)
      pltpu.sync_copy(rrows_v, rv_out.at[pl.ds(h * 16, 16)])

  def zb_body(i, carry):
    zbuf_v[i, :] = jnp.zeros((L,), jnp.float32)
    return carry
  lax.fori_loop(0, ZR, zb_body, 0)

  for t in range(2):
    q = 2 * cid + t

    def z_body(z, carry):
      pltpu.sync_copy(zbuf_v, acc.at[pl.ds(sid * RPS + z * ZR, ZR)])
      return carry
    lax.fori_loop(0, RPS // ZR, z_body, 0)
    plsc.subcore_barrier()

    def chunk_body(j, carry):
      r0 = sid * RPT + j * NB
      pltpu.sync_copy(src4q_hbm.at[q, pl.ds(r0, NB)], srcq_v)
      pltpu.sync_copy(dst_hbm.at[pl.ds(r0, NB)], dst_v)
      gds = [None] * NB
      sds = [None] * NB
      for jj in range(NB + LG):
        if jj < NB:
          slot = jj % R
          if jj >= R:
            sds[jj - R].wait()
          gds[jj] = pltpu.async_copy(g4_hbm.at[srcq_v.at[jj]],
                                     ring_v.at[slot], gsem.at[slot])
        if jj >= LG:
          k = jj - LG
          gds[k].wait()
          sds[k] = pltpu.async_copy(ring_v.at[k % R],
                                    acc.at[dst_v.at[k]],
                                    ssem.at[k % R], add=True)
      for k in range(NB - R, NB):
        sds[k].wait()
      return carry
    lax.fori_loop(0, NCH, chunk_body, 0)
    plsc.subcore_barrier()

    pltpu.sync_copy(acc.at[pl.ds(sid * RPS, RPS)],
                    s_out.at[pl.ds(sid * RPS, RPS), pl.ds(q * L, L)])


_conv_kernel = functools.partial(
    pl.kernel,
    out_type=[
        jax.ShapeDtypeStruct((NP, 4 * L), jnp.float32),
        jax.ShapeDtypeStruct((G, 64), jnp.float32),
    ],
    mesh=_sc_mesh(),
    compiler_params=_SC_PARAMS,
    scratch_types=[
        pltpu.VMEM_SHARED((NP, L), jnp.float32),
        pltpu.VMEM((NB, BK), jnp.int32),
        pltpu.VMEM((NB, BK), jnp.int32),
        pltpu.VMEM((R, BK, L), jnp.float32),
        pltpu.VMEM((ZR, L), jnp.float32),
        pltpu.VMEM((G,), jnp.int32),
        pltpu.VMEM((16, 64), jnp.float32),
        pltpu.SemaphoreType.DMA((R,)),
        pltpu.SemaphoreType.DMA((R,)),
    ],
)(_conv_body)


def _tca1_body(x_ref, w1, b1, w2, b2, w3, b3, x1_ref):
  xb = x_ref[...]
  h = xb * w1[...] + b1[...]
  h = jnp.dot(h, w2[...], precision="highest") + b2[...]
  h = jnp.dot(h, w3[...], precision="highest") + b3[...]
  x1_ref[...] = h


def _tca2_body(x1_ref, deg_ref, wc1, g1_ref, dinv_ref):
  deg = deg_ref[0] + deg_ref[1] + 1.0
  dinv = lax.rsqrt(deg)
  dinv_ref[...] = dinv
  g1_ref[...] = jnp.dot(x1_ref[...], wc1[...], precision="highest") * dinv


def _full(shape):
  return pl.BlockSpec(shape, lambda i: tuple(0 for _ in shape))


def _tca1(x, W1r, b1r, W2, b2r, W3, b3r):
  return pl.pallas_call(
      _tca1_body,
      grid=(NGRID,),
      in_specs=[
          pl.BlockSpec((NBLK, 1), lambda i: (i, 0)),
          _full((1, 32)), _full((1, 32)),
          _full((32, 128)), _full((1, 128)),
          _full((128, 32)), _full((1, 32)),
      ],
      out_specs=pl.BlockSpec((NBLK, 32), lambda i: (i, 0)),
      out_shape=jax.ShapeDtypeStruct((NP, 32), jnp.float32),
  )(x, W1r, b1r, W2, b2r, W3, b3r)


def _tca2(x1, deg2, Wc1):
  return pl.pallas_call(
      _tca2_body,
      grid=(NGRID,),
      in_specs=[
          pl.BlockSpec((NBLK, 32), lambda i: (i, 0)),
          pl.BlockSpec((NC, NBLK, 1), lambda i: (0, i, 0)),
          _full((32, 64)),
      ],
      out_specs=[
          pl.BlockSpec((NBLK, 64), lambda i: (i, 0)),
          pl.BlockSpec((NBLK, 1), lambda i: (i, 0)),
      ],
      out_shape=[
          jax.ShapeDtypeStruct((NP, 64), jnp.float32),
          jax.ShapeDtypeStruct((NP, 1), jnp.float32),
      ],
  )(x1, deg2, Wc1)


def _tcb_body(s1_ref, g1_ref, dinv_ref, bc1, rv1_ref, batch_ref,
              wc2a, wc2b, x2_ref, g2_ref):
  dinv = dinv_ref[...]
  x2 = dinv * (s1_ref[...] + g1_ref[...]) + bc1[...]
  x2_ref[...] = x2
  oh = (batch_ref[...] == jnp.arange(G, dtype=jnp.int32)[None, :]
        ).astype(jnp.float32)
  rext = jnp.dot(oh, rv1_ref[...], precision="highest")
  h2 = (jnp.dot(jnp.maximum(x2, 0.0), wc2a[...], precision="highest")
        + jnp.dot(jnp.maximum(rext, 0.0), wc2b[...], precision="highest"))
  g2_ref[...] = h2 * dinv


def _tcb(s1, g1, dinv, bc1r, rv1, batch2, Wc2a, Wc2b):
  return pl.pallas_call(
      _tcb_body,
      grid=(NGRID,),
      in_specs=[
          pl.BlockSpec((NBLK, 64), lambda i: (i, 0)),
          pl.BlockSpec((NBLK, 64), lambda i: (i, 0)),
          pl.BlockSpec((NBLK, 1), lambda i: (i, 0)),
          _full((1, 64)),
          _full((G, 32)),
          pl.BlockSpec((NBLK, 1), lambda i: (i, 0)),
          _full((64, 64)), _full((32, 64)),
      ],
      out_specs=[
          pl.BlockSpec((NBLK, 64), lambda i: (i, 0)),
          pl.BlockSpec((NBLK, 64), lambda i: (i, 0)),
      ],
      out_shape=[
          jax.ShapeDtypeStruct((NP, 64), jnp.float32),
          jax.ShapeDtypeStruct((NP, 64), jnp.float32),
      ],
  )(s1, g1, dinv, bc1r, rv1, batch2, Wc2a, Wc2b)


def _tcc_body(s2_ref, g2_ref, dinv_ref, bc2, rv2_ref, batch_ref,
              out_ref, seg_acc, cnt_acc):
  i = pl.program_id(0)
  x3 = jnp.maximum(dinv_ref[...] * (s2_ref[...] + g2_ref[...]) + bc2[...],
                   0.0)
  oh = (batch_ref[...] == jnp.arange(G, dtype=jnp.int32)[None, :]
        ).astype(jnp.float32)
  rext = jnp.dot(oh, rv2_ref[...], precision="highest")
  xc = jnp.concatenate([x3, rext], axis=1)
  part = lax.dot_general(oh, xc, (((0,), (0,)), ((), ())),
                         precision="highest")
  cntp = lax.dot_general(oh, jnp.ones((NBLK, 1), jnp.float32),
                         (((0,), (0,)), ((), ())), precision="highest")

  @pl.when(i == 0)
  def _init():
    seg_acc[...] = part
    cnt_acc[...] = cntp

  @pl.when(i > 0)
  def _accum():
    seg_acc[...] = seg_acc[...] + part
    cnt_acc[...] = cnt_acc[...] + cntp

  @pl.when(i == NGRID - 1)
  def _final():
    out_ref[...] = seg_acc[...] / jnp.maximum(cnt_acc[...], 1.0)


def _tcc(s2, g2, dinv, bc2r, rv2, batch2):
  return pl.pallas_call(
      _tcc_body,
      grid=(NGRID,),
      in_specs=[
          pl.BlockSpec((NBLK, 64), lambda i: (i, 0)),
          pl.BlockSpec((NBLK, 64), lambda i: (i, 0)),
          pl.BlockSpec((NBLK, 1), lambda i: (i, 0)),
          _full((1, 64)),
          _full((G, 64)),
          pl.BlockSpec((NBLK, 1), lambda i: (i, 0)),
      ],
      out_specs=pl.BlockSpec((G, 2 * 64), lambda i: (0, 0)),
      out_shape=jax.ShapeDtypeStruct((G, 2 * 64), jnp.float32),
      scratch_shapes=[
          pltpu.VMEM((G, 2 * 64), jnp.float32),
          pltpu.VMEM((G, 1), jnp.float32),
      ],
  )(s2, g2, dinv, bc2r, rv2, batch2)


def kernel(x, edge_index, batch, root_index, W1, b1, W2, b2, W3, b3,
           Wc1, bc1, Wc2, bc2):
  x = x.astype(jnp.float32)
  src = edge_index[0]
  dst = edge_index[1]

  xp = jnp.pad(x, ((0, NP - N), (0, 0)))
  batchp = jnp.pad(batch, (0, NP - N), constant_values=G).reshape(NP, 1)
  srcp = jnp.pad(src, (0, EP - E))
  dstp = jnp.pad(dst, (0, EP - E), constant_values=N)
  src4q = (srcp * 4)[None, :] + jnp.arange(4, dtype=jnp.int32)[:, None]
  src4q = src4q.reshape(4, EROWS, BK)
  dst2 = dstp.reshape(EROWS, BK)
  W1r, b1r = W1.reshape(1, 32), b1.reshape(1, 32)
  b2r, b3r = b2.reshape(1, 128), b3.reshape(1, 32)
  bc1r, bc2r = bc1.reshape(1, 64), bc2.reshape(1, 64)
  Wc2a, Wc2b = Wc2[:64], Wc2[64:]

  deg2, _ = _deg_kernel(dst)
  x1 = _tca1(xp, W1r, b1r, W2, b2r, W3, b3r)
  g1, dinv = _tca2(x1, deg2.reshape(NC, NP, 1), Wc1)

  x1w = jnp.pad(x1, ((0, 0), (0, 32)))
  s1, rv1 = _conv_kernel(g1.reshape(4 * NP, L), src4q, dst2,
                         root_index, x1w)
  x2, g2 = _tcb(s1, g1, dinv, bc1r, rv1[:, :32], batchp, Wc2a, Wc2b)

  s2, rv2 = _conv_kernel(g2.reshape(4 * NP, L), src4q, dst2,
                         root_index, x2)
  return _tcc(s2, g2, dinv, bc2r, rv2, batchp)
